# Initial kernel scaffold; baseline (speedup 1.0000x reference)
import jax
import jax.numpy as jnp
from jax.experimental import pallas as pl


def kernel(x_list, edge_index, ix, _, W1_0, b1_0, W2_0, b2_0, W1_1, b1_1, W2_1, b2_1):
    raise NotImplementedError("write your pallas kernel here")


# speedup vs baseline: 3.6927x; 3.6927x over previous
import functools

import jax
import jax.numpy as jnp
from jax import lax
from jax.experimental import pallas as pl
from jax.experimental.pallas import tpu as pltpu
from jax.experimental.pallas import tpu_sc as plsc

N = 10000
E = 320000
D = 128
H = 512

NC = 2
NS = 16
NW = NC * NS
CHUNK = 128
E_TOT = E + N
CH_PER_TILE = 88
E_PAD = NW * CH_PER_TILE * CHUNK
N_ACC = 10240
ROWS_PER_TILE = N_ACC // NS
OUT_CH = ROWS_PER_TILE // CHUNK

_sc_mesh = plsc.VectorSubcoreMesh(core_axis_name="c", subcore_axis_name="s")


@functools.partial(
    pl.kernel,
    mesh=_sc_mesh,
    out_type=jax.ShapeDtypeStruct((NC, N_ACC, D), jnp.float32),
    scratch_types=[
        pltpu.VMEM_SHARED((N_ACC, D), jnp.float32),
        pltpu.VMEM((CHUNK, D), jnp.float32),
        pltpu.VMEM((CH_PER_TILE, CHUNK), jnp.int32),
        pltpu.VMEM((CH_PER_TILE, CHUNK), jnp.int32),
        pltpu.SemaphoreType.DMA,
    ],
)
def _raw_agg(x_hbm, src_hbm, dst_hbm, zeros_hbm, out_hbm, acc, rows, src_v,
             dst_v, sem):
    c = lax.axis_index("c")
    s = lax.axis_index("s")
    row0 = s * ROWS_PER_TILE
    cbase = (c * NS + s) * CH_PER_TILE

    pltpu.sync_copy(zeros_hbm, rows)

    def zbody(i, carry):
        pltpu.sync_copy(rows, acc.at[pl.ds(row0 + i * CHUNK, CHUNK)])
        return carry

    lax.fori_loop(0, OUT_CH, zbody, 0)

    pltpu.sync_copy(src_hbm.at[pl.ds(cbase, CH_PER_TILE)], src_v)
    pltpu.sync_copy(dst_hbm.at[pl.ds(cbase, CH_PER_TILE)], dst_v)
    plsc.subcore_barrier()

    def ebody(j, carry):
        pltpu.async_copy(x_hbm.at[src_v.at[j]], rows, sem).wait()
        pltpu.sync_copy(rows, acc.at[dst_v.at[j]], add=True)
        return carry

    lax.fori_loop(0, CH_PER_TILE, ebody, 0)
    plsc.subcore_barrier()

    def obody(i, carry):
        pltpu.sync_copy(acc.at[pl.ds(row0 + i * CHUNK, CHUNK)], rows)
        pltpu.sync_copy(rows, out_hbm.at[c, pl.ds(row0 + i * CHUNK, CHUNK)])
        return carry

    lax.fori_loop(0, OUT_CH, obody, 0)


@functools.partial(
    pl.kernel,
    mesh=_sc_mesh,
    out_type=jax.ShapeDtypeStruct((NC, N_ACC, 16), jnp.float32),
    scratch_types=[
        pltpu.VMEM_SHARED((N_ACC, 16), jnp.float32),
        pltpu.VMEM((CHUNK, 16), jnp.float32),
        pltpu.VMEM((CH_PER_TILE, CHUNK), jnp.int32),
    ],
)
def _hist(dst_hbm, ones_hbm, zeros_hbm, out_hbm, acc, rows, dst_v):
    c = lax.axis_index("c")
    s = lax.axis_index("s")
    row0 = s * ROWS_PER_TILE
    cbase = (c * NS + s) * CH_PER_TILE

    pltpu.sync_copy(zeros_hbm, rows)

    def zbody(i, carry):
        pltpu.sync_copy(rows, acc.at[pl.ds(row0 + i * CHUNK, CHUNK)])
        return carry

    lax.fori_loop(0, OUT_CH, zbody, 0)

    pltpu.sync_copy(dst_hbm.at[pl.ds(cbase, CH_PER_TILE)], dst_v)
    pltpu.sync_copy(ones_hbm, rows)
    plsc.subcore_barrier()

    def ebody(j, carry):
        pltpu.sync_copy(rows, acc.at[dst_v.at[j]], add=True)
        return carry

    lax.fori_loop(0, CH_PER_TILE, ebody, 0)
    plsc.subcore_barrier()

    def obody(i, carry):
        pltpu.sync_copy(acc.at[pl.ds(row0 + i * CHUNK, CHUNK)], rows)
        pltpu.sync_copy(rows, out_hbm.at[c, pl.ds(row0 + i * CHUNK, CHUNK)])
        return carry

    lax.fori_loop(0, OUT_CH, obody, 0)


BLK = 400
GRID = N // BLK


def _dis_of(deg_ref):
    d = deg_ref[0, :, :1] + deg_ref[1, :, :1]
    return lax.rsqrt(d)


def _t0_body(x_ref, deg_ref, o_ref):
    o_ref[...] = x_ref[...] * _dis_of(deg_ref)


def _mlp_body(r_ref, deg_ref, w1_ref, b1_ref, w2_ref, o_ref):
    dis = _dis_of(deg_ref)
    px = (r_ref[0] + r_ref[1]) * dis
    h = jnp.dot(px, w1_ref[...], preferred_element_type=jnp.float32,
                precision=lax.Precision.HIGHEST)
    h = jnp.maximum(h + b1_ref[...], 0.0)
    y = jnp.dot(h, w2_ref[...], preferred_element_type=jnp.float32,
                precision=lax.Precision.HIGHEST)
    o_ref[...] = y * dis


def _t2_body(r_ref, deg_ref, b_ref, o_ref):
    dis = _dis_of(deg_ref)
    o_ref[...] = ((r_ref[0] + r_ref[1]) * dis + b_ref[...]) * dis


def _t4_body(r_ref, deg_ref, b_ref, o_ref):
    dis = _dis_of(deg_ref)
    o_ref[...] = (r_ref[0] + r_ref[1]) * dis + b_ref[...]


_r_spec = pl.BlockSpec((NC, BLK, D), lambda i: (0, i, 0))
_deg_spec = pl.BlockSpec((NC, BLK, 16), lambda i: (0, i, 0))
_x_spec = pl.BlockSpec((BLK, D), lambda i: (i, 0))
_out_spec = pl.BlockSpec((BLK, D), lambda i: (i, 0))
_out_shape = jax.ShapeDtypeStruct((N, D), jnp.float32)


def _full(shape):
    return pl.BlockSpec(shape, lambda i: tuple(0 for _ in shape))


_t0 = pl.pallas_call(
    _t0_body, grid=(GRID,),
    in_specs=[_x_spec, _deg_spec],
    out_specs=_out_spec, out_shape=_out_shape)

_mlp = pl.pallas_call(
    _mlp_body, grid=(GRID,),
    in_specs=[_r_spec, _deg_spec, _full((D, H)), _full((1, H)), _full((H, D))],
    out_specs=_out_spec, out_shape=_out_shape)

_t2 = pl.pallas_call(
    _t2_body, grid=(GRID,),
    in_specs=[_r_spec, _deg_spec, _full((1, D))],
    out_specs=_out_spec, out_shape=_out_shape)

_t4 = pl.pallas_call(
    _t4_body, grid=(GRID,),
    in_specs=[_r_spec, _deg_spec, _full((1, D))],
    out_specs=_out_spec, out_shape=_out_shape)


def kernel(x_list, edge_index, ix, _, W1_0, b1_0, W2_0, b2_0, W1_1, b1_1,
           W2_1, b2_1):
    ei = edge_index.astype(jnp.int32)
    loop = jnp.arange(N, dtype=jnp.int32)
    src = jnp.concatenate([ei[0], loop])
    dst = jnp.concatenate([ei[1], loop])
    pad = E_PAD - E_TOT
    src = jnp.concatenate([src, jnp.zeros((pad,), jnp.int32)])
    dst = jnp.concatenate([dst, jnp.full((pad,), N, jnp.int32)])
    src = src.reshape(E_PAD // CHUNK, CHUNK)
    dst = dst.reshape(E_PAD // CHUNK, CHUNK)

    zeros128 = jnp.zeros((CHUNK, D), jnp.float32)
    ones16 = jnp.ones((CHUNK, 16), jnp.float32)
    zeros16 = jnp.zeros((CHUNK, 16), jnp.float32)

    deg = _hist(dst, ones16, zeros16)
    u = _t0(x_list, deg)
    r = _raw_agg(u, src, dst, zeros128)
    u = _mlp(r, deg, W1_0, b1_0.reshape(1, H), W2_0)
    r = _raw_agg(u, src, dst, zeros128)
    u = _t2(r, deg, b2_0.reshape(1, D))
    r = _raw_agg(u, src, dst, zeros128)
    u = _mlp(r, deg, W1_1, b1_1.reshape(1, H), W2_1)
    r = _raw_agg(u, src, dst, zeros128)
    return _t4(r, deg, b2_1.reshape(1, D))
